# baseline (speedup 1.0000x reference)
import functools

import jax
import jax.numpy as jnp
from jax import lax
from jax.experimental import pallas as pl
from jax.experimental.pallas import tpu as pltpu
from jax.experimental.pallas import tpu_sc as plsc

BN_EPS_ = 1e-5
_CH = 128
_TILES = 16


def _edge_loop(x_hbm, src_hbm, dst_hbm, idxs_v, idxd_v, rows, acc_sh, sems,
               base0, nchunks):
    gsem, ssems = sems[0], sems[1:]

    def scat_wait(k):
        pltpu.make_async_copy(rows[k], acc_sh.at[idxd_v.at[k]],
                              ssems[k]).wait()

    @pl.loop(0, nchunks, step=2)
    def _(i):
        for k in range(2):
            @pl.when(i >= 2)
            def _():
                scat_wait(k)
            base = base0 + (i + k) * _CH
            pltpu.sync_copy(src_hbm.at[pl.ds(base, _CH)], idxs_v)
            pltpu.sync_copy(dst_hbm.at[pl.ds(base, _CH)], idxd_v.at[k])
            pltpu.async_copy(x_hbm.at[idxs_v], rows[k], gsem).wait()
            pltpu.async_copy(rows[k], acc_sh.at[idxd_v.at[k]], ssems[k],
                             add=True)

    for k in range(2):
        scat_wait(k)


def _sc_mesh():
    return plsc.VectorSubcoreMesh(core_axis_name="c", subcore_axis_name="s")


def _agg_edge_split(x, zeros, src, dst, ept):
    n, f = x.shape
    rpt = n // _TILES

    @functools.partial(
        pl.kernel,
        out_type=(
            jax.ShapeDtypeStruct((n, f), jnp.float32),
            jax.ShapeDtypeStruct((n, f), jnp.float32),
        ),
        mesh=_sc_mesh(),
        scratch_types=[
            pltpu.VMEM((_CH,), jnp.int32),
            pltpu.VMEM((2, _CH), jnp.int32),
            pltpu.VMEM((_CH, f), jnp.float32),
            pltpu.VMEM((_CH, f), jnp.float32),
            pltpu.VMEM_SHARED((n, f), jnp.float32),
            pltpu.SemaphoreType.DMA,
            pltpu.SemaphoreType.DMA,
            pltpu.SemaphoreType.DMA,
        ],
    )
    def agg_kernel(x_hbm, z_hbm, src_hbm, dst_hbm, o0_hbm, o1_hbm,
                   idxs_v, idxd_v, rows_a, rows_b, acc_sh, *sems):
        rows = (rows_a, rows_b)
        c = lax.axis_index("c")
        s = lax.axis_index("s")

        def run(init_hbm, o_hbm):
            pltpu.sync_copy(init_hbm.at[pl.ds(s * rpt, rpt)],
                            acc_sh.at[pl.ds(s * rpt, rpt)])
            plsc.subcore_barrier()
            _edge_loop(x_hbm, src_hbm, dst_hbm, idxs_v, idxd_v, rows,
                       acc_sh, sems, (c * _TILES + s) * ept, ept // _CH)
            plsc.subcore_barrier()
            pltpu.sync_copy(acc_sh.at[pl.ds(s * rpt, rpt)],
                            o_hbm.at[pl.ds(s * rpt, rpt)])

        @pl.when(c == 0)
        def _():
            run(x_hbm, o0_hbm)

        @pl.when(c == 1)
        def _():
            run(z_hbm, o1_hbm)

    return agg_kernel(x, zeros, src, dst)


def _agg_feat_split(x_lo, x_hi, src, dst, ept):
    n, fh = x_lo.shape
    rpt = n // _TILES

    @functools.partial(
        pl.kernel,
        out_type=(
            jax.ShapeDtypeStruct((n, fh), jnp.float32),
            jax.ShapeDtypeStruct((n, fh), jnp.float32),
        ),
        mesh=_sc_mesh(),
        scratch_types=[
            pltpu.VMEM((_CH,), jnp.int32),
            pltpu.VMEM((2, _CH), jnp.int32),
            pltpu.VMEM((_CH, fh), jnp.float32),
            pltpu.VMEM((_CH, fh), jnp.float32),
            pltpu.VMEM_SHARED((n, fh), jnp.float32),
            pltpu.SemaphoreType.DMA,
            pltpu.SemaphoreType.DMA,
            pltpu.SemaphoreType.DMA,
        ],
    )
    def agg_kernel(xlo_hbm, xhi_hbm, src_hbm, dst_hbm, olo_hbm, ohi_hbm,
                   idxs_v, idxd_v, rows_a, rows_b, acc_sh, *sems):
        rows = (rows_a, rows_b)
        c = lax.axis_index("c")
        s = lax.axis_index("s")

        def run(x_hbm, o_hbm):
            pltpu.sync_copy(x_hbm.at[pl.ds(s * rpt, rpt)],
                            acc_sh.at[pl.ds(s * rpt, rpt)])
            plsc.subcore_barrier()
            _edge_loop(x_hbm, src_hbm, dst_hbm, idxs_v, idxd_v, rows,
                       acc_sh, sems, s * ept, ept // _CH)
            plsc.subcore_barrier()
            pltpu.sync_copy(acc_sh.at[pl.ds(s * rpt, rpt)],
                            o_hbm.at[pl.ds(s * rpt, rpt)])

        @pl.when(c == 0)
        def _():
            run(xlo_hbm, olo_hbm)

        @pl.when(c == 1)
        def _():
            run(xhi_hbm, ohi_hbm)

    return agg_kernel(x_lo, x_hi, src, dst)


def _mlp_tc(a_lo, a_hi, w1a, w1b, b1, w2, b2, sum_inputs, split_out):
    n = a_lo.shape[0]
    kh = a_lo.shape[1]
    h = w2.shape[1]
    blk = 1264
    hiprec = lax.Precision.HIGHEST

    def body(alo_ref, ahi_ref, w1a_ref, w1b_ref, b1_ref, w2_ref, b2_ref,
             *out_refs):
        if sum_inputs:
            t = jnp.dot(alo_ref[...] + ahi_ref[...], w1a_ref[...],
                        preferred_element_type=jnp.float32, precision=hiprec)
        else:
            t = jnp.dot(alo_ref[...], w1a_ref[...],
                        preferred_element_type=jnp.float32, precision=hiprec)
            t += jnp.dot(ahi_ref[...], w1b_ref[...],
                         preferred_element_type=jnp.float32, precision=hiprec)
        t = jnp.maximum(t + b1_ref[...], 0.0)
        o = jnp.dot(t, w2_ref[...],
                    preferred_element_type=jnp.float32, precision=hiprec)
        o = jnp.maximum(o + b2_ref[...], 0.0)
        if split_out:
            out_refs[0][...] = o[:, : h // 2]
            out_refs[1][...] = o[:, h // 2:]
        else:
            out_refs[0][...] = o

    if split_out:
        out_shape = (
            jax.ShapeDtypeStruct((n, h // 2), jnp.float32),
            jax.ShapeDtypeStruct((n, h // 2), jnp.float32),
        )
        out_specs = (
            pl.BlockSpec((blk, h // 2), lambda i: (i, 0)),
            pl.BlockSpec((blk, h // 2), lambda i: (i, 0)),
        )
    else:
        out_shape = jax.ShapeDtypeStruct((n, h), jnp.float32)
        out_specs = pl.BlockSpec((blk, h), lambda i: (i, 0))

    return pl.pallas_call(
        body,
        grid=(n // blk,),
        in_specs=[
            pl.BlockSpec((blk, kh), lambda i: (i, 0)),
            pl.BlockSpec((blk, kh), lambda i: (i, 0)),
            pl.BlockSpec(w1a.shape, lambda i: (0, 0)),
            pl.BlockSpec(w1b.shape, lambda i: (0, 0)),
            pl.BlockSpec((1, h), lambda i: (0, 0)),
            pl.BlockSpec((h, h), lambda i: (0, 0)),
            pl.BlockSpec((1, h), lambda i: (0, 0)),
        ],
        out_specs=out_specs,
        out_shape=out_shape,
    )(a_lo, a_hi, w1a, w1b, b1, w2, b2)


def kernel(x, edge_index, W0_1, b0_1, g0, be0, W0_2, b0_2,
           W1_1, b1_1, g1, be1, W1_2, b1_2):
    n, d = x.shape
    h = W0_1.shape[1]
    e = edge_index.shape[1]

    src = edge_index[0].astype(jnp.int32)
    dst = edge_index[1].astype(jnp.int32)

    ept0 = -(-e // (2 * _TILES * 2 * _CH)) * 2 * _CH
    ept1 = 2 * ept0
    e_pad = ept0 * 2 * _TILES
    npad = -(-n // (_TILES * 8)) * (_TILES * 8)
    if e_pad != e:
        src = jnp.concatenate([src, jnp.zeros((e_pad - e,), jnp.int32)])
        dst = jnp.concatenate([dst, jnp.full((e_pad - e,), n, jnp.int32)])

    s0 = g0 / jnp.sqrt(1.0 + BN_EPS_)
    w0s = W0_1 * s0[None, :]
    b0f = (b0_1 * s0 + be0).reshape(1, h)
    s1 = g1 / jnp.sqrt(1.0 + BN_EPS_)
    w1s = W1_1 * s1[None, :]
    b1f = (b1_1 * s1 + be1).reshape(1, h)
    b0_2r = b0_2.reshape(1, h)
    b1_2r = b1_2.reshape(1, h)

    xp = jnp.pad(x, ((0, npad - n), (0, 0)))
    zp = jnp.zeros_like(xp)
    p0, p1 = _agg_edge_split(xp, zp, src, dst, ept0)
    h_lo, h_hi = _mlp_tc(p0, p1, w0s, w0s, b0f, W0_2, b0_2r,
                         sum_inputs=True, split_out=True)

    a1_lo, a1_hi = _agg_feat_split(h_lo, h_hi, src, dst, ept1)
    out = _mlp_tc(a1_lo, a1_hi, w1s[: h // 2], w1s[h // 2:],
                  b1f, W1_2, b1_2r, sum_inputs=False, split_out=False)
    return out[:n]


# speedup vs baseline: 1.9767x; 1.9767x over previous
import functools

import jax
import jax.numpy as jnp
from jax import lax
from jax.experimental import pallas as pl
from jax.experimental.pallas import tpu as pltpu
from jax.experimental.pallas import tpu_sc as plsc

BN_EPS_ = 1e-5
_CH = 128
_TILES = 16


def _edge_loop(x_hbm, src_hbm, dst_hbm, idxs_v, idxd_v, rows, acc_sh, sems,
               base0, nchunks):
    gsem, ssems = sems[0], sems[1:]

    def scat_wait(k):
        pltpu.make_async_copy(rows[k], acc_sh.at[idxd_v.at[k]],
                              ssems[k]).wait()

    @pl.loop(0, nchunks, step=2)
    def _(i):
        for k in range(2):
            @pl.when(i >= 2)
            def _():
                scat_wait(k)
            base = base0 + (i + k) * _CH
            pltpu.sync_copy(src_hbm.at[pl.ds(base, _CH)], idxs_v)
            pltpu.sync_copy(dst_hbm.at[pl.ds(base, _CH)], idxd_v.at[k])
            pltpu.async_copy(x_hbm.at[idxs_v], rows[k], gsem).wait()
            pltpu.async_copy(rows[k], acc_sh.at[idxd_v.at[k]], ssems[k],
                             add=True)

    for k in range(2):
        scat_wait(k)


def _sc_mesh():
    return plsc.VectorSubcoreMesh(core_axis_name="c", subcore_axis_name="s")


def _agg_edge_split(x, zeros, src, dst, ept):
    n, f = x.shape
    rpt = n // _TILES

    @functools.partial(
        pl.kernel,
        out_type=(
            jax.ShapeDtypeStruct((n, f), jnp.float32),
            jax.ShapeDtypeStruct((n, f), jnp.float32),
        ),
        mesh=_sc_mesh(),
        scratch_types=[
            pltpu.VMEM((_CH,), jnp.int32),
            pltpu.VMEM((2, _CH), jnp.int32),
            pltpu.VMEM((_CH, f), jnp.float32),
            pltpu.VMEM((_CH, f), jnp.float32),
            pltpu.VMEM_SHARED((n, f), jnp.float32),
            pltpu.SemaphoreType.DMA,
            pltpu.SemaphoreType.DMA,
            pltpu.SemaphoreType.DMA,
        ],
    )
    def agg_kernel(x_hbm, z_hbm, src_hbm, dst_hbm, o0_hbm, o1_hbm,
                   idxs_v, idxd_v, rows_a, rows_b, acc_sh, *sems):
        rows = (rows_a, rows_b)
        c = lax.axis_index("c")
        s = lax.axis_index("s")

        def run(init_hbm, o_hbm):
            pltpu.sync_copy(init_hbm.at[pl.ds(s * rpt, rpt)],
                            acc_sh.at[pl.ds(s * rpt, rpt)])
            plsc.subcore_barrier()
            _edge_loop(x_hbm, src_hbm, dst_hbm, idxs_v, idxd_v, rows,
                       acc_sh, sems, (c * _TILES + s) * ept, ept // _CH)
            plsc.subcore_barrier()
            pltpu.sync_copy(acc_sh.at[pl.ds(s * rpt, rpt)],
                            o_hbm.at[pl.ds(s * rpt, rpt)])

        @pl.when(c == 0)
        def _():
            run(x_hbm, o0_hbm)

        @pl.when(c == 1)
        def _():
            run(z_hbm, o1_hbm)

    return agg_kernel(x, zeros, src, dst)


def _agg_feat_split(x_lo, x_hi, src, dst, ept):
    n, fh = x_lo.shape
    rpt = n // _TILES

    @functools.partial(
        pl.kernel,
        out_type=(
            jax.ShapeDtypeStruct((n, fh), jnp.float32),
            jax.ShapeDtypeStruct((n, fh), jnp.float32),
        ),
        mesh=_sc_mesh(),
        scratch_types=[
            pltpu.VMEM((_CH,), jnp.int32),
            pltpu.VMEM((2, _CH), jnp.int32),
            pltpu.VMEM((_CH, fh), jnp.float32),
            pltpu.VMEM((_CH, fh), jnp.float32),
            pltpu.VMEM_SHARED((n, fh), jnp.float32),
            pltpu.SemaphoreType.DMA,
            pltpu.SemaphoreType.DMA,
            pltpu.SemaphoreType.DMA,
        ],
    )
    def agg_kernel(xlo_hbm, xhi_hbm, src_hbm, dst_hbm, olo_hbm, ohi_hbm,
                   idxs_v, idxd_v, rows_a, rows_b, acc_sh, *sems):
        rows = (rows_a, rows_b)
        c = lax.axis_index("c")
        s = lax.axis_index("s")

        def run(x_hbm, o_hbm):
            pltpu.sync_copy(x_hbm.at[pl.ds(s * rpt, rpt)],
                            acc_sh.at[pl.ds(s * rpt, rpt)])
            plsc.subcore_barrier()
            _edge_loop(x_hbm, src_hbm, dst_hbm, idxs_v, idxd_v, rows,
                       acc_sh, sems, s * ept, ept // _CH)
            plsc.subcore_barrier()
            pltpu.sync_copy(acc_sh.at[pl.ds(s * rpt, rpt)],
                            o_hbm.at[pl.ds(s * rpt, rpt)])

        @pl.when(c == 0)
        def _():
            run(xlo_hbm, olo_hbm)

        @pl.when(c == 1)
        def _():
            run(xhi_hbm, ohi_hbm)

    return agg_kernel(x_lo, x_hi, src, dst)


def _mlp_tc(a_lo, a_hi, w1a, w1b, b1, w2, b2, sum_inputs, split_out):
    n = a_lo.shape[0]
    kh = a_lo.shape[1]
    h = w2.shape[1]
    blk = 1264
    hiprec = lax.Precision.HIGHEST

    def body(alo_ref, ahi_ref, w1a_ref, w1b_ref, b1_ref, w2_ref, b2_ref,
             *out_refs):
        if sum_inputs:
            t = jnp.dot(alo_ref[...] + ahi_ref[...], w1a_ref[...],
                        preferred_element_type=jnp.float32, precision=hiprec)
        else:
            t = jnp.dot(alo_ref[...], w1a_ref[...],
                        preferred_element_type=jnp.float32, precision=hiprec)
            t += jnp.dot(ahi_ref[...], w1b_ref[...],
                         preferred_element_type=jnp.float32, precision=hiprec)
        t = jnp.maximum(t + b1_ref[...], 0.0)
        o = jnp.dot(t, w2_ref[...],
                    preferred_element_type=jnp.float32, precision=hiprec)
        o = jnp.maximum(o + b2_ref[...], 0.0)
        if split_out:
            out_refs[0][...] = o[:, : h // 2]
            out_refs[1][...] = o[:, h // 2:]
        else:
            out_refs[0][...] = o

    if split_out:
        out_shape = (
            jax.ShapeDtypeStruct((n, h // 2), jnp.float32),
            jax.ShapeDtypeStruct((n, h // 2), jnp.float32),
        )
        out_specs = (
            pl.BlockSpec((blk, h // 2), lambda i: (i, 0)),
            pl.BlockSpec((blk, h // 2), lambda i: (i, 0)),
        )
    else:
        out_shape = jax.ShapeDtypeStruct((n, h), jnp.float32)
        out_specs = pl.BlockSpec((blk, h), lambda i: (i, 0))

    return pl.pallas_call(
        body,
        grid=(n // blk,),
        in_specs=[
            pl.BlockSpec((blk, kh), lambda i: (i, 0)),
            pl.BlockSpec((blk, kh), lambda i: (i, 0)),
            pl.BlockSpec(w1a.shape, lambda i: (0, 0)),
            pl.BlockSpec(w1b.shape, lambda i: (0, 0)),
            pl.BlockSpec((1, h), lambda i: (0, 0)),
            pl.BlockSpec((h, h), lambda i: (0, 0)),
            pl.BlockSpec((1, h), lambda i: (0, 0)),
        ],
        out_specs=out_specs,
        out_shape=out_shape,
    )(a_lo, a_hi, w1a, w1b, b1, w2, b2)


def kernel(x, edge_index, W0_1, b0_1, g0, be0, W0_2, b0_2,
           W1_1, b1_1, g1, be1, W1_2, b1_2):
    n, d = x.shape
    h = W0_1.shape[1]
    e = edge_index.shape[1]

    src = edge_index[0].astype(jnp.int32)
    dst = edge_index[1].astype(jnp.int32)

    ept0 = -(-e // (2 * _TILES * 2 * _CH)) * 2 * _CH
    ept1 = 2 * ept0
    e_pad = ept0 * 2 * _TILES
    npad = -(-n // (_TILES * 8)) * (_TILES * 8)
    if e_pad != e:
        npadidx = jnp.arange(e_pad - e, dtype=jnp.int32)
        src = jnp.concatenate([src, npadidx % n])
        dst = jnp.concatenate([dst, n + npadidx % (npad - n)])

    s0 = g0 / jnp.sqrt(1.0 + BN_EPS_)
    w0s = W0_1 * s0[None, :]
    b0f = (b0_1 * s0 + be0).reshape(1, h)
    s1 = g1 / jnp.sqrt(1.0 + BN_EPS_)
    w1s = W1_1 * s1[None, :]
    b1f = (b1_1 * s1 + be1).reshape(1, h)
    b0_2r = b0_2.reshape(1, h)
    b1_2r = b1_2.reshape(1, h)

    xp = jnp.pad(x, ((0, npad - n), (0, 0)))
    zp = jnp.zeros_like(xp)
    p0, p1 = _agg_edge_split(xp, zp, src, dst, ept0)
    h_lo, h_hi = _mlp_tc(p0, p1, w0s, w0s, b0f, W0_2, b0_2r,
                         sum_inputs=True, split_out=True)

    a1_lo, a1_hi = _agg_feat_split(h_lo, h_hi, src, dst, ept1)
    out = _mlp_tc(a1_lo, a1_hi, w1s[: h // 2], w1s[h // 2:],
                  b1f, W1_2, b1_2r, sum_inputs=False, split_out=False)
    return out[:n]


# speedup vs baseline: 2.5090x; 1.2693x over previous
import functools

import jax
import jax.numpy as jnp
from jax import lax
from jax.experimental import pallas as pl
from jax.experimental.pallas import tpu as pltpu
from jax.experimental.pallas import tpu_sc as plsc

BN_EPS_ = 1e-5
_CH = 128
_TILES = 16


def _edge_loop(x_hbm, src_hbm, dst_hbm, idxs_v, idxd_v, rows, acc_sh, sems,
               base0, nchunks):
    gsems, ssems = sems[:2], sems[2:]

    def scat_wait(k):
        pltpu.make_async_copy(rows[k], acc_sh.at[idxd_v.at[k]],
                              ssems[k]).wait()

    def retire(j):
        pltpu.make_async_copy(x_hbm.at[idxs_v.at[j]], rows[j],
                              gsems[j]).wait()
        pltpu.async_copy(rows[j], acc_sh.at[idxd_v.at[j]], ssems[j],
                         add=True)

    @pl.loop(0, nchunks, step=2)
    def _(i):
        for k in range(2):
            @pl.when(i >= 2)
            def _():
                scat_wait(k)
            base = base0 + (i + k) * _CH
            pltpu.sync_copy(src_hbm.at[pl.ds(base, _CH)], idxs_v.at[k])
            pltpu.sync_copy(dst_hbm.at[pl.ds(base, _CH)], idxd_v.at[k])
            pltpu.async_copy(x_hbm.at[idxs_v.at[k]], rows[k], gsems[k])
            if k == 0:
                @pl.when(i >= 2)
                def _():
                    retire(1)
            else:
                retire(0)

    retire(1)
    for k in range(2):
        scat_wait(k)


def _sc_mesh():
    return plsc.VectorSubcoreMesh(core_axis_name="c", subcore_axis_name="s")


def _agg_edge_split(x, zeros, src, dst, ept):
    n, f = x.shape
    rpt = n // _TILES

    @functools.partial(
        pl.kernel,
        out_type=(
            jax.ShapeDtypeStruct((n, f), jnp.float32),
            jax.ShapeDtypeStruct((n, f), jnp.float32),
        ),
        mesh=_sc_mesh(),
        scratch_types=[
            pltpu.VMEM((2, _CH), jnp.int32),
            pltpu.VMEM((2, _CH), jnp.int32),
            pltpu.VMEM((_CH, f), jnp.float32),
            pltpu.VMEM((_CH, f), jnp.float32),
            pltpu.VMEM_SHARED((n, f), jnp.float32),
            pltpu.SemaphoreType.DMA,
            pltpu.SemaphoreType.DMA,
            pltpu.SemaphoreType.DMA,
            pltpu.SemaphoreType.DMA,
        ],
    )
    def agg_kernel(x_hbm, z_hbm, src_hbm, dst_hbm, o0_hbm, o1_hbm,
                   idxs_v, idxd_v, rows_a, rows_b, acc_sh, *sems):
        rows = (rows_a, rows_b)
        c = lax.axis_index("c")
        s = lax.axis_index("s")

        def run(init_hbm, o_hbm):
            pltpu.sync_copy(init_hbm.at[pl.ds(s * rpt, rpt)],
                            acc_sh.at[pl.ds(s * rpt, rpt)])
            plsc.subcore_barrier()
            _edge_loop(x_hbm, src_hbm, dst_hbm, idxs_v, idxd_v, rows,
                       acc_sh, sems, (c * _TILES + s) * ept, ept // _CH)
            plsc.subcore_barrier()
            pltpu.sync_copy(acc_sh.at[pl.ds(s * rpt, rpt)],
                            o_hbm.at[pl.ds(s * rpt, rpt)])

        @pl.when(c == 0)
        def _():
            run(x_hbm, o0_hbm)

        @pl.when(c == 1)
        def _():
            run(z_hbm, o1_hbm)

    return agg_kernel(x, zeros, src, dst)


def _agg_feat_split(x_lo, x_hi, src, dst, ept):
    n, fh = x_lo.shape
    rpt = n // _TILES

    @functools.partial(
        pl.kernel,
        out_type=(
            jax.ShapeDtypeStruct((n, fh), jnp.float32),
            jax.ShapeDtypeStruct((n, fh), jnp.float32),
        ),
        mesh=_sc_mesh(),
        scratch_types=[
            pltpu.VMEM((2, _CH), jnp.int32),
            pltpu.VMEM((2, _CH), jnp.int32),
            pltpu.VMEM((_CH, fh), jnp.float32),
            pltpu.VMEM((_CH, fh), jnp.float32),
            pltpu.VMEM_SHARED((n, fh), jnp.float32),
            pltpu.SemaphoreType.DMA,
            pltpu.SemaphoreType.DMA,
            pltpu.SemaphoreType.DMA,
            pltpu.SemaphoreType.DMA,
        ],
    )
    def agg_kernel(xlo_hbm, xhi_hbm, src_hbm, dst_hbm, olo_hbm, ohi_hbm,
                   idxs_v, idxd_v, rows_a, rows_b, acc_sh, *sems):
        rows = (rows_a, rows_b)
        c = lax.axis_index("c")
        s = lax.axis_index("s")

        def run(x_hbm, o_hbm):
            pltpu.sync_copy(x_hbm.at[pl.ds(s * rpt, rpt)],
                            acc_sh.at[pl.ds(s * rpt, rpt)])
            plsc.subcore_barrier()
            _edge_loop(x_hbm, src_hbm, dst_hbm, idxs_v, idxd_v, rows,
                       acc_sh, sems, s * ept, ept // _CH)
            plsc.subcore_barrier()
            pltpu.sync_copy(acc_sh.at[pl.ds(s * rpt, rpt)],
                            o_hbm.at[pl.ds(s * rpt, rpt)])

        @pl.when(c == 0)
        def _():
            run(xlo_hbm, olo_hbm)

        @pl.when(c == 1)
        def _():
            run(xhi_hbm, ohi_hbm)

    return agg_kernel(x_lo, x_hi, src, dst)


def _mlp_tc(a_lo, a_hi, w1a, w1b, b1, w2, b2, sum_inputs, split_out):
    n = a_lo.shape[0]
    kh = a_lo.shape[1]
    h = w2.shape[1]
    blk = 1264
    hiprec = lax.Precision.HIGHEST

    def body(alo_ref, ahi_ref, w1a_ref, w1b_ref, b1_ref, w2_ref, b2_ref,
             *out_refs):
        if sum_inputs:
            t = jnp.dot(alo_ref[...] + ahi_ref[...], w1a_ref[...],
                        preferred_element_type=jnp.float32, precision=hiprec)
        else:
            t = jnp.dot(alo_ref[...], w1a_ref[...],
                        preferred_element_type=jnp.float32, precision=hiprec)
            t += jnp.dot(ahi_ref[...], w1b_ref[...],
                         preferred_element_type=jnp.float32, precision=hiprec)
        t = jnp.maximum(t + b1_ref[...], 0.0)
        o = jnp.dot(t, w2_ref[...],
                    preferred_element_type=jnp.float32, precision=hiprec)
        o = jnp.maximum(o + b2_ref[...], 0.0)
        if split_out:
            out_refs[0][...] = o[:, : h // 2]
            out_refs[1][...] = o[:, h // 2:]
        else:
            out_refs[0][...] = o

    if split_out:
        out_shape = (
            jax.ShapeDtypeStruct((n, h // 2), jnp.float32),
            jax.ShapeDtypeStruct((n, h // 2), jnp.float32),
        )
        out_specs = (
            pl.BlockSpec((blk, h // 2), lambda i: (i, 0)),
            pl.BlockSpec((blk, h // 2), lambda i: (i, 0)),
        )
    else:
        out_shape = jax.ShapeDtypeStruct((n, h), jnp.float32)
        out_specs = pl.BlockSpec((blk, h), lambda i: (i, 0))

    return pl.pallas_call(
        body,
        grid=(n // blk,),
        in_specs=[
            pl.BlockSpec((blk, kh), lambda i: (i, 0)),
            pl.BlockSpec((blk, kh), lambda i: (i, 0)),
            pl.BlockSpec(w1a.shape, lambda i: (0, 0)),
            pl.BlockSpec(w1b.shape, lambda i: (0, 0)),
            pl.BlockSpec((1, h), lambda i: (0, 0)),
            pl.BlockSpec((h, h), lambda i: (0, 0)),
            pl.BlockSpec((1, h), lambda i: (0, 0)),
        ],
        out_specs=out_specs,
        out_shape=out_shape,
    )(a_lo, a_hi, w1a, w1b, b1, w2, b2)


def kernel(x, edge_index, W0_1, b0_1, g0, be0, W0_2, b0_2,
           W1_1, b1_1, g1, be1, W1_2, b1_2):
    n, d = x.shape
    h = W0_1.shape[1]
    e = edge_index.shape[1]

    src = edge_index[0].astype(jnp.int32)
    dst = edge_index[1].astype(jnp.int32)

    ept0 = -(-e // (2 * _TILES * 2 * _CH)) * 2 * _CH
    ept1 = 2 * ept0
    e_pad = ept0 * 2 * _TILES
    npad = -(-n // (_TILES * 8)) * (_TILES * 8)
    if e_pad != e:
        npadidx = jnp.arange(e_pad - e, dtype=jnp.int32)
        src = jnp.concatenate([src, npadidx % n])
        dst = jnp.concatenate([dst, n + npadidx % (npad - n)])

    s0 = g0 / jnp.sqrt(1.0 + BN_EPS_)
    w0s = W0_1 * s0[None, :]
    b0f = (b0_1 * s0 + be0).reshape(1, h)
    s1 = g1 / jnp.sqrt(1.0 + BN_EPS_)
    w1s = W1_1 * s1[None, :]
    b1f = (b1_1 * s1 + be1).reshape(1, h)
    b0_2r = b0_2.reshape(1, h)
    b1_2r = b1_2.reshape(1, h)

    xp = jnp.pad(x, ((0, npad - n), (0, 0)))
    zp = jnp.zeros_like(xp)
    p0, p1 = _agg_edge_split(xp, zp, src, dst, ept0)
    h_lo, h_hi = _mlp_tc(p0, p1, w0s, w0s, b0f, W0_2, b0_2r,
                         sum_inputs=True, split_out=True)

    a1_lo, a1_hi = _agg_feat_split(h_lo, h_hi, src, dst, ept1)
    out = _mlp_tc(a1_lo, a1_hi, w1s[: h // 2], w1s[h // 2:],
                  b1f, W1_2, b1_2r, sum_inputs=False, split_out=False)
    return out[:n]


# speedup vs baseline: 2.9367x; 1.1705x over previous
import functools

import jax
import jax.numpy as jnp
from jax import lax
from jax.experimental import pallas as pl
from jax.experimental.pallas import tpu as pltpu
from jax.experimental.pallas import tpu_sc as plsc

BN_EPS_ = 1e-5
_CH = 128
_TILES = 16


def _edge_loop(x_hbm, src_hbm, dst_hbm, idxs_v, idxd_v, rows, acc_sh, sems,
               base0, nchunks):
    gsems, ssems, isem = sems[:2], sems[2:4], sems[4]

    def scat_wait(k):
        pltpu.make_async_copy(rows[k], acc_sh.at[idxd_v.at[k]],
                              ssems[k]).wait()

    def retire(j):
        pltpu.make_async_copy(x_hbm.at[idxs_v.at[j]], rows[j],
                              gsems[j]).wait()
        pltpu.async_copy(rows[j], acc_sh.at[idxd_v.at[j]], ssems[j],
                         add=True)

    @pl.loop(0, nchunks, step=2)
    def _(i):
        for k in range(2):
            @pl.when(i >= 2)
            def _():
                scat_wait(k)
            base = base0 + (i + k) * _CH
            pltpu.async_copy(src_hbm.at[pl.ds(base, _CH)], idxs_v.at[k],
                             isem)
            pltpu.async_copy(dst_hbm.at[pl.ds(base, _CH)], idxd_v.at[k],
                             isem)
            pltpu.make_async_copy(src_hbm.at[pl.ds(base, _CH)],
                                  idxs_v.at[k], isem).wait()
            pltpu.make_async_copy(dst_hbm.at[pl.ds(base, _CH)],
                                  idxd_v.at[k], isem).wait()
            pltpu.async_copy(x_hbm.at[idxs_v.at[k]], rows[k], gsems[k])
            if k == 0:
                @pl.when(i >= 2)
                def _():
                    retire(1)
            else:
                retire(0)

    retire(1)
    for k in range(2):
        scat_wait(k)


def _sc_mesh():
    return plsc.VectorSubcoreMesh(core_axis_name="c", subcore_axis_name="s")


def _agg_edge_split(x, zeros, src, dst, ept):
    n, f = x.shape
    rpt = n // _TILES

    @functools.partial(
        pl.kernel,
        out_type=(
            jax.ShapeDtypeStruct((n, f), jnp.float32),
            jax.ShapeDtypeStruct((n, f), jnp.float32),
        ),
        mesh=_sc_mesh(),
        scratch_types=[
            pltpu.VMEM((2, _CH), jnp.int32),
            pltpu.VMEM((2, _CH), jnp.int32),
            pltpu.VMEM((_CH, f), jnp.float32),
            pltpu.VMEM((_CH, f), jnp.float32),
            pltpu.VMEM_SHARED((n, f), jnp.float32),
            pltpu.SemaphoreType.DMA,
            pltpu.SemaphoreType.DMA,
            pltpu.SemaphoreType.DMA,
            pltpu.SemaphoreType.DMA,
            pltpu.SemaphoreType.DMA,
        ],
    )
    def agg_kernel(x_hbm, z_hbm, src_hbm, dst_hbm, o0_hbm, o1_hbm,
                   idxs_v, idxd_v, rows_a, rows_b, acc_sh, *sems):
        rows = (rows_a, rows_b)
        c = lax.axis_index("c")
        s = lax.axis_index("s")

        def run(init_hbm, o_hbm):
            pltpu.sync_copy(init_hbm.at[pl.ds(s * rpt, rpt)],
                            acc_sh.at[pl.ds(s * rpt, rpt)])
            plsc.subcore_barrier()
            _edge_loop(x_hbm, src_hbm, dst_hbm, idxs_v, idxd_v, rows,
                       acc_sh, sems, (c * _TILES + s) * ept, ept // _CH)
            plsc.subcore_barrier()
            pltpu.sync_copy(acc_sh.at[pl.ds(s * rpt, rpt)],
                            o_hbm.at[pl.ds(s * rpt, rpt)])

        @pl.when(c == 0)
        def _():
            run(x_hbm, o0_hbm)

        @pl.when(c == 1)
        def _():
            run(z_hbm, o1_hbm)

    return agg_kernel(x, zeros, src, dst)


def _agg_feat_split(x_lo, x_hi, src, dst, ept):
    n, fh = x_lo.shape
    rpt = n // _TILES

    @functools.partial(
        pl.kernel,
        out_type=(
            jax.ShapeDtypeStruct((n, fh), jnp.float32),
            jax.ShapeDtypeStruct((n, fh), jnp.float32),
        ),
        mesh=_sc_mesh(),
        scratch_types=[
            pltpu.VMEM((2, _CH), jnp.int32),
            pltpu.VMEM((2, _CH), jnp.int32),
            pltpu.VMEM((_CH, fh), jnp.float32),
            pltpu.VMEM((_CH, fh), jnp.float32),
            pltpu.VMEM_SHARED((n, fh), jnp.float32),
            pltpu.SemaphoreType.DMA,
            pltpu.SemaphoreType.DMA,
            pltpu.SemaphoreType.DMA,
            pltpu.SemaphoreType.DMA,
            pltpu.SemaphoreType.DMA,
        ],
    )
    def agg_kernel(xlo_hbm, xhi_hbm, src_hbm, dst_hbm, olo_hbm, ohi_hbm,
                   idxs_v, idxd_v, rows_a, rows_b, acc_sh, *sems):
        rows = (rows_a, rows_b)
        c = lax.axis_index("c")
        s = lax.axis_index("s")

        def run(x_hbm, o_hbm):
            pltpu.sync_copy(x_hbm.at[pl.ds(s * rpt, rpt)],
                            acc_sh.at[pl.ds(s * rpt, rpt)])
            plsc.subcore_barrier()
            _edge_loop(x_hbm, src_hbm, dst_hbm, idxs_v, idxd_v, rows,
                       acc_sh, sems, s * ept, ept // _CH)
            plsc.subcore_barrier()
            pltpu.sync_copy(acc_sh.at[pl.ds(s * rpt, rpt)],
                            o_hbm.at[pl.ds(s * rpt, rpt)])

        @pl.when(c == 0)
        def _():
            run(xlo_hbm, olo_hbm)

        @pl.when(c == 1)
        def _():
            run(xhi_hbm, ohi_hbm)

    return agg_kernel(x_lo, x_hi, src, dst)


def _mlp_tc(a_lo, a_hi, w1a, w1b, b1, w2, b2, sum_inputs, split_out):
    n = a_lo.shape[0]
    kh = a_lo.shape[1]
    h = w2.shape[1]
    blk = 1264
    hiprec = lax.Precision.HIGHEST

    def body(alo_ref, ahi_ref, w1a_ref, w1b_ref, b1_ref, w2_ref, b2_ref,
             *out_refs):
        if sum_inputs:
            t = jnp.dot(alo_ref[...] + ahi_ref[...], w1a_ref[...],
                        preferred_element_type=jnp.float32, precision=hiprec)
        else:
            t = jnp.dot(alo_ref[...], w1a_ref[...],
                        preferred_element_type=jnp.float32, precision=hiprec)
            t += jnp.dot(ahi_ref[...], w1b_ref[...],
                         preferred_element_type=jnp.float32, precision=hiprec)
        t = jnp.maximum(t + b1_ref[...], 0.0)
        o = jnp.dot(t, w2_ref[...],
                    preferred_element_type=jnp.float32, precision=hiprec)
        o = jnp.maximum(o + b2_ref[...], 0.0)
        if split_out:
            out_refs[0][...] = o[:, : h // 2]
            out_refs[1][...] = o[:, h // 2:]
        else:
            out_refs[0][...] = o

    if split_out:
        out_shape = (
            jax.ShapeDtypeStruct((n, h // 2), jnp.float32),
            jax.ShapeDtypeStruct((n, h // 2), jnp.float32),
        )
        out_specs = (
            pl.BlockSpec((blk, h // 2), lambda i: (i, 0)),
            pl.BlockSpec((blk, h // 2), lambda i: (i, 0)),
        )
    else:
        out_shape = jax.ShapeDtypeStruct((n, h), jnp.float32)
        out_specs = pl.BlockSpec((blk, h), lambda i: (i, 0))

    return pl.pallas_call(
        body,
        grid=(n // blk,),
        in_specs=[
            pl.BlockSpec((blk, kh), lambda i: (i, 0)),
            pl.BlockSpec((blk, kh), lambda i: (i, 0)),
            pl.BlockSpec(w1a.shape, lambda i: (0, 0)),
            pl.BlockSpec(w1b.shape, lambda i: (0, 0)),
            pl.BlockSpec((1, h), lambda i: (0, 0)),
            pl.BlockSpec((h, h), lambda i: (0, 0)),
            pl.BlockSpec((1, h), lambda i: (0, 0)),
        ],
        out_specs=out_specs,
        out_shape=out_shape,
    )(a_lo, a_hi, w1a, w1b, b1, w2, b2)


def kernel(x, edge_index, W0_1, b0_1, g0, be0, W0_2, b0_2,
           W1_1, b1_1, g1, be1, W1_2, b1_2):
    n, d = x.shape
    h = W0_1.shape[1]
    e = edge_index.shape[1]

    src = edge_index[0].astype(jnp.int32)
    dst = edge_index[1].astype(jnp.int32)

    ept0 = -(-e // (2 * _TILES * 2 * _CH)) * 2 * _CH
    ept1 = 2 * ept0
    e_pad = ept0 * 2 * _TILES
    npad = -(-n // (_TILES * 8)) * (_TILES * 8)
    if e_pad != e:
        npadidx = jnp.arange(e_pad - e, dtype=jnp.int32)
        src = jnp.concatenate([src, npadidx % n])
        dst = jnp.concatenate([dst, n + npadidx % (npad - n)])

    s0 = g0 / jnp.sqrt(1.0 + BN_EPS_)
    w0s = W0_1 * s0[None, :]
    b0f = (b0_1 * s0 + be0).reshape(1, h)
    s1 = g1 / jnp.sqrt(1.0 + BN_EPS_)
    w1s = W1_1 * s1[None, :]
    b1f = (b1_1 * s1 + be1).reshape(1, h)
    b0_2r = b0_2.reshape(1, h)
    b1_2r = b1_2.reshape(1, h)

    xp = jnp.pad(x, ((0, npad - n), (0, 0)))
    zp = jnp.zeros_like(xp)
    p0, p1 = _agg_edge_split(xp, zp, src, dst, ept0)
    h_lo, h_hi = _mlp_tc(p0, p1, w0s, w0s, b0f, W0_2, b0_2r,
                         sum_inputs=True, split_out=True)

    a1_lo, a1_hi = _agg_feat_split(h_lo, h_hi, src, dst, ept1)
    out = _mlp_tc(a1_lo, a1_hi, w1s[: h // 2], w1s[h // 2:],
                  b1f, W1_2, b1_2r, sum_inputs=False, split_out=False)
    return out[:n]


# speedup vs baseline: 3.2518x; 1.1073x over previous
import functools

import jax
import jax.numpy as jnp
from jax import lax
from jax.experimental import pallas as pl
from jax.experimental.pallas import tpu as pltpu
from jax.experimental.pallas import tpu_sc as plsc

BN_EPS_ = 1e-5
_CH = 128
_TILES = 16


def _edge_loop(x_hbm, src_hbm, dst_hbm, idxs_v, idxd_v, rows, acc_sh, sems,
               base0, nchunks):
    gsems, ssems, isems = sems[:2], sems[2:4], sems[4:]

    def idx_fetch(ci, m):
        base = base0 + ci * _CH
        pltpu.async_copy(src_hbm.at[pl.ds(base, _CH)], idxs_v.at[m],
                         isems[m])
        pltpu.async_copy(dst_hbm.at[pl.ds(base, _CH)], idxd_v.at[m],
                         isems[m])

    def idx_wait(m):
        pltpu.make_async_copy(src_hbm.at[pl.ds(base0, _CH)], idxs_v.at[m],
                              isems[m]).wait()
        pltpu.make_async_copy(dst_hbm.at[pl.ds(base0, _CH)], idxd_v.at[m],
                              isems[m]).wait()

    def scat_wait(k):
        pltpu.make_async_copy(rows[k], acc_sh.at[idxd_v.at[0]],
                              ssems[k]).wait()

    def retire(j, m):
        pltpu.make_async_copy(x_hbm.at[idxs_v.at[m]], rows[j],
                              gsems[j]).wait()
        pltpu.async_copy(rows[j], acc_sh.at[idxd_v.at[m]], ssems[j],
                         add=True)

    idx_fetch(0, 0)
    idx_fetch(1, 1)

    @pl.loop(0, nchunks, step=4)
    def _(i):
        for t in range(4):
            k = t % 2
            if t < 2:
                @pl.when(i >= 4)
                def _():
                    scat_wait(k)
                idx_fetch(i + t + 2, t + 2)
            else:
                scat_wait(k)

                @pl.when(i + 4 < nchunks)
                def _():
                    idx_fetch(i + t + 2, t - 2)
            idx_wait(t)
            pltpu.async_copy(x_hbm.at[idxs_v.at[t]], rows[k], gsems[k])
            if t == 0:
                @pl.when(i >= 4)
                def _():
                    retire(1, 3)
            else:
                retire(k ^ 1, t - 1)

    retire(1, 3)
    scat_wait(0)
    scat_wait(1)


def _sc_mesh():
    return plsc.VectorSubcoreMesh(core_axis_name="c", subcore_axis_name="s")


def _agg_edge_split(x, zeros, src, dst, ept):
    n, f = x.shape
    rpt = n // _TILES

    @functools.partial(
        pl.kernel,
        out_type=(
            jax.ShapeDtypeStruct((n, f), jnp.float32),
            jax.ShapeDtypeStruct((n, f), jnp.float32),
        ),
        mesh=_sc_mesh(),
        scratch_types=[
            pltpu.VMEM((4, _CH), jnp.int32),
            pltpu.VMEM((4, _CH), jnp.int32),
            pltpu.VMEM((_CH, f), jnp.float32),
            pltpu.VMEM((_CH, f), jnp.float32),
            pltpu.VMEM_SHARED((n, f), jnp.float32),
            pltpu.SemaphoreType.DMA,
            pltpu.SemaphoreType.DMA,
            pltpu.SemaphoreType.DMA,
            pltpu.SemaphoreType.DMA,
            pltpu.SemaphoreType.DMA,
            pltpu.SemaphoreType.DMA,
            pltpu.SemaphoreType.DMA,
            pltpu.SemaphoreType.DMA,
        ],
    )
    def agg_kernel(x_hbm, z_hbm, src_hbm, dst_hbm, o0_hbm, o1_hbm,
                   idxs_v, idxd_v, rows_a, rows_b, acc_sh, *sems):
        rows = (rows_a, rows_b)
        c = lax.axis_index("c")
        s = lax.axis_index("s")

        def run(init_hbm, o_hbm):
            pltpu.sync_copy(init_hbm.at[pl.ds(s * rpt, rpt)],
                            acc_sh.at[pl.ds(s * rpt, rpt)])
            plsc.subcore_barrier()
            _edge_loop(x_hbm, src_hbm, dst_hbm, idxs_v, idxd_v, rows,
                       acc_sh, sems, (c * _TILES + s) * ept, ept // _CH)
            plsc.subcore_barrier()
            pltpu.sync_copy(acc_sh.at[pl.ds(s * rpt, rpt)],
                            o_hbm.at[pl.ds(s * rpt, rpt)])

        @pl.when(c == 0)
        def _():
            run(x_hbm, o0_hbm)

        @pl.when(c == 1)
        def _():
            run(z_hbm, o1_hbm)

    return agg_kernel(x, zeros, src, dst)


def _agg_feat_split(x_lo, x_hi, src, dst, ept):
    n, fh = x_lo.shape
    rpt = n // _TILES

    @functools.partial(
        pl.kernel,
        out_type=(
            jax.ShapeDtypeStruct((n, fh), jnp.float32),
            jax.ShapeDtypeStruct((n, fh), jnp.float32),
        ),
        mesh=_sc_mesh(),
        scratch_types=[
            pltpu.VMEM((4, _CH), jnp.int32),
            pltpu.VMEM((4, _CH), jnp.int32),
            pltpu.VMEM((_CH, fh), jnp.float32),
            pltpu.VMEM((_CH, fh), jnp.float32),
            pltpu.VMEM_SHARED((n, fh), jnp.float32),
            pltpu.SemaphoreType.DMA,
            pltpu.SemaphoreType.DMA,
            pltpu.SemaphoreType.DMA,
            pltpu.SemaphoreType.DMA,
            pltpu.SemaphoreType.DMA,
            pltpu.SemaphoreType.DMA,
            pltpu.SemaphoreType.DMA,
            pltpu.SemaphoreType.DMA,
        ],
    )
    def agg_kernel(xlo_hbm, xhi_hbm, src_hbm, dst_hbm, olo_hbm, ohi_hbm,
                   idxs_v, idxd_v, rows_a, rows_b, acc_sh, *sems):
        rows = (rows_a, rows_b)
        c = lax.axis_index("c")
        s = lax.axis_index("s")

        def run(x_hbm, o_hbm):
            pltpu.sync_copy(x_hbm.at[pl.ds(s * rpt, rpt)],
                            acc_sh.at[pl.ds(s * rpt, rpt)])
            plsc.subcore_barrier()
            _edge_loop(x_hbm, src_hbm, dst_hbm, idxs_v, idxd_v, rows,
                       acc_sh, sems, s * ept, ept // _CH)
            plsc.subcore_barrier()
            pltpu.sync_copy(acc_sh.at[pl.ds(s * rpt, rpt)],
                            o_hbm.at[pl.ds(s * rpt, rpt)])

        @pl.when(c == 0)
        def _():
            run(xlo_hbm, olo_hbm)

        @pl.when(c == 1)
        def _():
            run(xhi_hbm, ohi_hbm)

    return agg_kernel(x_lo, x_hi, src, dst)


def _mlp_tc(a_lo, a_hi, w1a, w1b, b1, w2, b2, sum_inputs, split_out):
    n = a_lo.shape[0]
    kh = a_lo.shape[1]
    h = w2.shape[1]
    blk = 1264
    hiprec = lax.Precision.HIGHEST

    def body(alo_ref, ahi_ref, w1a_ref, w1b_ref, b1_ref, w2_ref, b2_ref,
             *out_refs):
        if sum_inputs:
            t = jnp.dot(alo_ref[...] + ahi_ref[...], w1a_ref[...],
                        preferred_element_type=jnp.float32, precision=hiprec)
        else:
            t = jnp.dot(alo_ref[...], w1a_ref[...],
                        preferred_element_type=jnp.float32, precision=hiprec)
            t += jnp.dot(ahi_ref[...], w1b_ref[...],
                         preferred_element_type=jnp.float32, precision=hiprec)
        t = jnp.maximum(t + b1_ref[...], 0.0)
        o = jnp.dot(t, w2_ref[...],
                    preferred_element_type=jnp.float32, precision=hiprec)
        o = jnp.maximum(o + b2_ref[...], 0.0)
        if split_out:
            out_refs[0][...] = o[:, : h // 2]
            out_refs[1][...] = o[:, h // 2:]
        else:
            out_refs[0][...] = o

    if split_out:
        out_shape = (
            jax.ShapeDtypeStruct((n, h // 2), jnp.float32),
            jax.ShapeDtypeStruct((n, h // 2), jnp.float32),
        )
        out_specs = (
            pl.BlockSpec((blk, h // 2), lambda i: (i, 0)),
            pl.BlockSpec((blk, h // 2), lambda i: (i, 0)),
        )
    else:
        out_shape = jax.ShapeDtypeStruct((n, h), jnp.float32)
        out_specs = pl.BlockSpec((blk, h), lambda i: (i, 0))

    return pl.pallas_call(
        body,
        grid=(n // blk,),
        in_specs=[
            pl.BlockSpec((blk, kh), lambda i: (i, 0)),
            pl.BlockSpec((blk, kh), lambda i: (i, 0)),
            pl.BlockSpec(w1a.shape, lambda i: (0, 0)),
            pl.BlockSpec(w1b.shape, lambda i: (0, 0)),
            pl.BlockSpec((1, h), lambda i: (0, 0)),
            pl.BlockSpec((h, h), lambda i: (0, 0)),
            pl.BlockSpec((1, h), lambda i: (0, 0)),
        ],
        out_specs=out_specs,
        out_shape=out_shape,
    )(a_lo, a_hi, w1a, w1b, b1, w2, b2)


def kernel(x, edge_index, W0_1, b0_1, g0, be0, W0_2, b0_2,
           W1_1, b1_1, g1, be1, W1_2, b1_2):
    n, d = x.shape
    h = W0_1.shape[1]
    e = edge_index.shape[1]

    src = edge_index[0].astype(jnp.int32)
    dst = edge_index[1].astype(jnp.int32)

    ept0 = -(-e // (2 * _TILES * 2 * _CH)) * 2 * _CH
    ept1 = 2 * ept0
    e_pad = ept0 * 2 * _TILES
    npad = -(-n // (_TILES * 8)) * (_TILES * 8)
    if e_pad != e:
        npadidx = jnp.arange(e_pad - e, dtype=jnp.int32)
        src = jnp.concatenate([src, npadidx % n])
        dst = jnp.concatenate([dst, n + npadidx % (npad - n)])

    s0 = g0 / jnp.sqrt(1.0 + BN_EPS_)
    w0s = W0_1 * s0[None, :]
    b0f = (b0_1 * s0 + be0).reshape(1, h)
    s1 = g1 / jnp.sqrt(1.0 + BN_EPS_)
    w1s = W1_1 * s1[None, :]
    b1f = (b1_1 * s1 + be1).reshape(1, h)
    b0_2r = b0_2.reshape(1, h)
    b1_2r = b1_2.reshape(1, h)

    xp = jnp.pad(x, ((0, npad - n), (0, 0)))
    zp = jnp.zeros_like(xp)
    p0, p1 = _agg_edge_split(xp, zp, src, dst, ept0)
    h_lo, h_hi = _mlp_tc(p0, p1, w0s, w0s, b0f, W0_2, b0_2r,
                         sum_inputs=True, split_out=True)

    a1_lo, a1_hi = _agg_feat_split(h_lo, h_hi, src, dst, ept1)
    out = _mlp_tc(a1_lo, a1_hi, w1s[: h // 2], w1s[h // 2:],
                  b1f, W1_2, b1_2r, sum_inputs=False, split_out=False)
    return out[:n]
